# baseline (speedup 1.0000x reference)
import functools

import jax
import jax.numpy as jnp
from jax import lax
from jax.experimental import pallas as pl
from jax.experimental.pallas import tpu as pltpu
from jax.experimental.pallas import tpu_sc as plsc

B = 4096
E = 200
EP = 256
H = 50
HP = 56
L = 16
NC, NS = 2, 16
NW = NC * NS
BPW = B // NW
NBLK = 13
NBIAS = 782
GM = 3.5


def _splat1(ref, i):
    return plsc.load_gather(ref, [jnp.full((L,), i, jnp.int32)])


def _svdpp_sc(user, movie, mr1, sq, iku, ikm, Bu2, Bi2, Pp, Qp, Yp):
    mesh = plsc.VectorSubcoreMesh(core_axis_name="c", subcore_axis_name="s")
    cp = pltpu.CompilerParams(
        needs_layout_passes=False, use_tc_tiling_on_sc=True
    )

    @functools.partial(
        pl.kernel,
        out_type=jax.ShapeDtypeStruct((B,), jnp.float32),
        mesh=mesh,
        compiler_params=cp,
        scratch_types=[
            pltpu.VMEM((BPW,), jnp.int32),
            pltpu.VMEM((BPW,), jnp.int32),
            pltpu.VMEM((BPW * HP,), jnp.int32),
            pltpu.VMEM((BPW,), jnp.float32),
            pltpu.VMEM((BPW,), jnp.float32),
            pltpu.VMEM((BPW,), jnp.float32),
            pltpu.VMEM((BPW,), jnp.int32),
            pltpu.VMEM((BPW, 128), jnp.float32),
            pltpu.VMEM((BPW,), jnp.float32),
            pltpu.VMEM((BPW,), jnp.float32),
            pltpu.VMEM((BPW, EP), jnp.float32),
            pltpu.VMEM((BPW, EP), jnp.float32),
            pltpu.VMEM((HP, EP), jnp.float32),
            pltpu.VMEM((HP, EP), jnp.float32),
            pltpu.VMEM((BPW,), jnp.float32),
            pltpu.SemaphoreType.DMA,
            pltpu.SemaphoreType.DMA,
            pltpu.SemaphoreType.DMA,
        ],
    )
    def kern(user_h, movie_h, mr_h, sq_h, iku_h, ikm_h, bu_h, bi_h, p_h,
             q_h, y_h, out_h, uidx, midx, mr_v, sq_v, iku_v, ikm_v, rowi,
             biasbuf, bu_val, bi_val, p_v, q_v, ybuf0, ybuf1, outv,
             sem_pre, sem0, sem1):
        wid = lax.axis_index("s") * NC + lax.axis_index("c")
        base = wid * BPW

        pltpu.sync_copy(user_h.at[pl.ds(base, BPW)], uidx)
        pltpu.sync_copy(movie_h.at[pl.ds(base, BPW)], midx)
        pltpu.sync_copy(mr_h.at[pl.ds(base * HP, BPW * HP)], mr_v)
        pltpu.sync_copy(sq_h.at[pl.ds(base, BPW)], sq_v)
        pltpu.sync_copy(iku_h.at[pl.ds(base, BPW)], iku_v)
        pltpu.sync_copy(ikm_h.at[pl.ds(base, BPW)], ikm_v)

        hp = pltpu.async_copy(p_h.at[uidx], p_v, sem_pre)
        hq = pltpu.async_copy(q_h.at[midx], q_v, sem_pre)

        pltpu.async_copy(y_h.at[mr_v.at[pl.ds(0, HP)]], ybuf0, sem0)

        lane = lax.iota(jnp.int32, L)
        zero = jnp.zeros((L,), jnp.float32)

        def bias_values(idx_v, table_h, val_v):
            @pl.loop(0, BPW, step=L)
            def _(t):
                rowi[pl.ds(t, L)] = lax.shift_right_logical(
                    idx_v[pl.ds(t, L)], 7
                )

            pltpu.async_copy(table_h.at[rowi], biasbuf, sem_pre).wait()

            @pl.loop(0, BPW, step=L)
            def _(t):
                lanes = jnp.bitwise_and(idx_v[pl.ds(t, L)], 127)
                val_v[pl.ds(t, L)] = plsc.load_gather(
                    biasbuf, [lane + t, lanes]
                )

        bias_values(uidx, bu_h, bu_val)
        bias_values(midx, bi_h, bi_val)

        hp.wait()
        hq.wait()

        def compute(b, ybuf):
            def row_body(h, accs):
                return tuple(
                    accs[j] + ybuf[h, pl.ds(16 * j, L)] for j in range(NBLK)
                )

            accs = lax.fori_loop(0, H, row_body, (zero,) * NBLK)

            iku_s = _splat1(iku_v, b)
            ikm_s = _splat1(ikm_v, b)
            sq_s = _splat1(sq_v, b)
            ysc = iku_s / sq_s
            tsum = zero
            for j in range(NBLK):
                pj = p_v[b, pl.ds(16 * j, L)]
                qj = q_v[b, pl.ds(16 * j, L)]
                tsum = tsum + qj * (pj * iku_s + accs[j] * ysc)
            dot = jnp.sum(tsum)
            bu_s = _splat1(bu_val, b)
            bi_s = _splat1(bi_val, b)
            r = ikm_s * jnp.full((L,), dot, jnp.float32) \
                + bi_s * ikm_s + bu_s * iku_s + GM
            plsc.store_scatter(
                outv, [jnp.full((L,), b, jnp.int32)], r, mask=(lane == 0)
            )

        @pl.loop(0, BPW, step=2)
        def _(g):
            pltpu.async_copy(
                y_h.at[mr_v.at[pl.ds((g + 1) * HP, HP)]], ybuf1, sem1
            )
            pltpu.make_async_copy(
                y_h.at[mr_v.at[pl.ds(g * HP, HP)]], ybuf0, sem0
            ).wait()
            compute(g, ybuf0)

            @pl.when(g + 2 < BPW)
            def _():
                pltpu.async_copy(
                    y_h.at[mr_v.at[pl.ds((g + 2) * HP, HP)]], ybuf0, sem0
                )

            pltpu.make_async_copy(
                y_h.at[mr_v.at[pl.ds((g + 1) * HP, HP)]], ybuf1, sem1
            ).wait()
            compute(g + 1, ybuf1)

        pltpu.sync_copy(outv, out_h.at[pl.ds(base, BPW)])

    return kern(user, movie, mr1, sq, iku, ikm, Bu2, Bi2, Pp, Qp, Yp)


@jax.jit
def kernel(user, movie, movies_rated_by_this_user, users_who_rated_this_movie,
           sqrt_of_number_of_movies_rated_by_this_user,
           sqrt_of_number_of_users_who_rated_this_movie,
           is_known_user, is_known_movie, Bu, Bi, P, Q, Y):
    del users_who_rated_this_movie, sqrt_of_number_of_users_who_rated_this_movie
    sq = sqrt_of_number_of_movies_rated_by_this_user.reshape(B)
    iku = is_known_user.reshape(B)
    ikm = is_known_movie.reshape(B)

    Pp = jnp.pad(P, ((0, 0), (0, EP - E)))
    Qp = jnp.pad(Q, ((0, 0), (0, EP - E)))
    Yp = jnp.pad(Y, ((0, 0), (0, EP - E)))

    mr1 = jnp.pad(
        movies_rated_by_this_user.astype(jnp.int32), ((0, 0), (0, HP - H))
    ).reshape(B * HP)

    Bu2 = jnp.pad(Bu.reshape(-1), (0, NBIAS * 128 - Bu.shape[0])).reshape(
        NBIAS, 128
    )
    Bi2 = jnp.pad(Bi.reshape(-1), (0, NBIAS * 128 - Bi.shape[0])).reshape(
        NBIAS, 128
    )

    return _svdpp_sc(
        user.astype(jnp.int32), movie.astype(jnp.int32), mr1,
        sq, iku, ikm, Bu2, Bi2, Pp, Qp, Yp,
    )


# speedup vs baseline: 1.2832x; 1.2832x over previous
import functools

import jax
import jax.numpy as jnp
from jax import lax
from jax.experimental import pallas as pl
from jax.experimental.pallas import tpu as pltpu
from jax.experimental.pallas import tpu_sc as plsc

B = 4096
E = 200
EP = 256
H = 50
HP = 52
L = 16
NC, NS = 2, 16
NW = NC * NS
BPW = B // NW
NB0 = 8
NB1 = 5
NBIAS = 782
GM = 3.5


def _splat1(ref, i):
    return plsc.load_gather(ref, [jnp.full((L,), i, jnp.int32)])


def _svdpp_sc(user2, movie2, uidx1, midx1, mr2, sq, iku, ikm, Bu2, Bi2,
              Pt, Qt, Yt):
    mesh = plsc.VectorSubcoreMesh(core_axis_name="c", subcore_axis_name="s")
    cp = pltpu.CompilerParams(
        needs_layout_passes=False, use_tc_tiling_on_sc=False
    )

    @functools.partial(
        pl.kernel,
        out_type=jax.ShapeDtypeStruct((B,), jnp.float32),
        mesh=mesh,
        compiler_params=cp,
        scratch_types=[
            pltpu.VMEM((2 * BPW,), jnp.int32),
            pltpu.VMEM((2 * BPW,), jnp.int32),
            pltpu.VMEM((BPW,), jnp.int32),
            pltpu.VMEM((BPW,), jnp.int32),
            pltpu.VMEM((BPW * 2 * HP,), jnp.int32),
            pltpu.VMEM((BPW,), jnp.float32),
            pltpu.VMEM((BPW,), jnp.float32),
            pltpu.VMEM((BPW,), jnp.float32),
            pltpu.VMEM((BPW,), jnp.int32),
            pltpu.VMEM((BPW, 128), jnp.float32),
            pltpu.VMEM((BPW,), jnp.float32),
            pltpu.VMEM((BPW,), jnp.float32),
            pltpu.VMEM((2 * BPW, 128), jnp.float32),
            pltpu.VMEM((2 * BPW, 128), jnp.float32),
            pltpu.VMEM((2 * HP, 128), jnp.float32),
            pltpu.VMEM((2 * HP, 128), jnp.float32),
            pltpu.VMEM((BPW,), jnp.float32),
            pltpu.SemaphoreType.DMA,
            pltpu.SemaphoreType.DMA,
            pltpu.SemaphoreType.DMA,
        ],
    )
    def kern(user2_h, movie2_h, uidx_h, midx_h, mr_h, sq_h, iku_h, ikm_h,
             bu_h, bi_h, p_h, q_h, y_h, out_h, u2idx, m2idx, uidx, midx,
             mr_v, sq_v, iku_v, ikm_v, rowi, biasbuf, bu_val, bi_val, p_v,
             q_v, ybuf0, ybuf1, outv, sem_pre, sem0, sem1):
        wid = lax.axis_index("s") * NC + lax.axis_index("c")
        base = wid * BPW

        pltpu.sync_copy(user2_h.at[pl.ds(2 * base, 2 * BPW)], u2idx)
        pltpu.sync_copy(movie2_h.at[pl.ds(2 * base, 2 * BPW)], m2idx)
        pltpu.sync_copy(uidx_h.at[pl.ds(base, BPW)], uidx)
        pltpu.sync_copy(midx_h.at[pl.ds(base, BPW)], midx)
        pltpu.sync_copy(mr_h.at[pl.ds(base * 2 * HP, BPW * 2 * HP)], mr_v)
        pltpu.sync_copy(sq_h.at[pl.ds(base, BPW)], sq_v)
        pltpu.sync_copy(iku_h.at[pl.ds(base, BPW)], iku_v)
        pltpu.sync_copy(ikm_h.at[pl.ds(base, BPW)], ikm_v)

        hp = pltpu.async_copy(p_h.at[u2idx], p_v, sem_pre)
        hq = pltpu.async_copy(q_h.at[m2idx], q_v, sem_pre)

        pltpu.async_copy(y_h.at[mr_v.at[pl.ds(0, 2 * HP)]], ybuf0, sem0)

        lane = lax.iota(jnp.int32, L)
        zero = jnp.zeros((L,), jnp.float32)

        def bias_values(idx_v, table_h, val_v):
            @pl.loop(0, BPW, step=L)
            def _(t):
                rowi[pl.ds(t, L)] = lax.shift_right_logical(
                    idx_v[pl.ds(t, L)], 7
                )

            pltpu.async_copy(table_h.at[rowi], biasbuf, sem_pre).wait()

            @pl.loop(0, BPW, step=L)
            def _(t):
                lanes = jnp.bitwise_and(idx_v[pl.ds(t, L)], 127)
                val_v[pl.ds(t, L)] = plsc.load_gather(
                    biasbuf, [lane + t, lanes]
                )

        bias_values(uidx, bu_h, bu_val)
        bias_values(midx, bi_h, bi_val)

        hp.wait()
        hq.wait()

        def compute(b, ybuf):
            def row_body(h, accs):
                even = tuple(
                    accs[j] + ybuf[2 * h, pl.ds(16 * j, L)]
                    for j in range(NB0)
                )
                odd = tuple(
                    accs[NB0 + j] + ybuf[2 * h + 1, pl.ds(16 * j, L)]
                    for j in range(NB1)
                )
                return even + odd

            accs = lax.fori_loop(0, H, row_body, (zero,) * (NB0 + NB1))

            iku_s = _splat1(iku_v, b)
            ikm_s = _splat1(ikm_v, b)
            sq_s = _splat1(sq_v, b)
            ysc = iku_s / sq_s
            tsum = zero
            for j in range(NB0):
                pj = p_v[2 * b, pl.ds(16 * j, L)]
                qj = q_v[2 * b, pl.ds(16 * j, L)]
                tsum = tsum + qj * (pj * iku_s + accs[j] * ysc)
            for j in range(NB1):
                pj = p_v[2 * b + 1, pl.ds(16 * j, L)]
                qj = q_v[2 * b + 1, pl.ds(16 * j, L)]
                tsum = tsum + qj * (pj * iku_s + accs[NB0 + j] * ysc)
            dot = jnp.sum(tsum)
            bu_s = _splat1(bu_val, b)
            bi_s = _splat1(bi_val, b)
            r = ikm_s * jnp.full((L,), dot, jnp.float32) \
                + bi_s * ikm_s + bu_s * iku_s + GM
            plsc.store_scatter(
                outv, [jnp.full((L,), b, jnp.int32)], r, mask=(lane == 0)
            )

        NH = 2 * HP

        @pl.loop(0, BPW, step=2)
        def _(g):
            pltpu.async_copy(
                y_h.at[mr_v.at[pl.ds((g + 1) * NH, NH)]], ybuf1, sem1
            )
            pltpu.make_async_copy(
                y_h.at[mr_v.at[pl.ds(g * NH, NH)]], ybuf0, sem0
            ).wait()
            compute(g, ybuf0)

            @pl.when(g + 2 < BPW)
            def _():
                pltpu.async_copy(
                    y_h.at[mr_v.at[pl.ds((g + 2) * NH, NH)]], ybuf0, sem0
                )

            pltpu.make_async_copy(
                y_h.at[mr_v.at[pl.ds((g + 1) * NH, NH)]], ybuf1, sem1
            ).wait()
            compute(g + 1, ybuf1)

        pltpu.sync_copy(outv, out_h.at[pl.ds(base, BPW)])

    return kern(user2, movie2, uidx1, midx1, mr2, sq, iku, ikm, Bu2, Bi2,
                Pt, Qt, Yt)


def _halfrows(table, s):
    n = table.shape[0]
    padded = jnp.pad(table, ((0, 0), (0, EP - E))) * s
    return padded.reshape(2 * n, 128)


@jax.jit
def kernel(user, movie, movies_rated_by_this_user, users_who_rated_this_movie,
           sqrt_of_number_of_movies_rated_by_this_user,
           sqrt_of_number_of_users_who_rated_this_movie,
           is_known_user, is_known_movie, Bu, Bi, P, Q, Y):
    del users_who_rated_this_movie, sqrt_of_number_of_users_who_rated_this_movie
    sq = sqrt_of_number_of_movies_rated_by_this_user.reshape(B)
    iku = is_known_user.reshape(B)
    ikm = is_known_movie.reshape(B)

    s = lax.optimization_barrier(jnp.float32(1.0))
    Pt = _halfrows(P, s)
    Qt = _halfrows(Q, s)
    Yt = _halfrows(Y, s)

    two = jnp.arange(2, dtype=jnp.int32)
    user_i = user.astype(jnp.int32)
    movie_i = movie.astype(jnp.int32)
    user2 = (2 * user_i[:, None] + two).reshape(2 * B)
    movie2 = (2 * movie_i[:, None] + two).reshape(2 * B)

    mr_i = movies_rated_by_this_user.astype(jnp.int32)
    mr_p = jnp.pad(mr_i, ((0, 0), (0, HP - H)))
    mr2 = (2 * mr_p[:, :, None] + two).reshape(B * 2 * HP)

    Bu2 = (jnp.pad(Bu.reshape(-1), (0, NBIAS * 128 - Bu.shape[0])) * s
           ).reshape(NBIAS, 128)
    Bi2 = (jnp.pad(Bi.reshape(-1), (0, NBIAS * 128 - Bi.shape[0])) * s
           ).reshape(NBIAS, 128)

    return _svdpp_sc(user2, movie2, user_i, movie_i, mr2, sq, iku, ikm,
                     Bu2, Bi2, Pt, Qt, Yt)


# speedup vs baseline: 1.2930x; 1.0076x over previous
import functools

import jax
import jax.numpy as jnp
from jax import lax
from jax.experimental import pallas as pl
from jax.experimental.pallas import tpu as pltpu
from jax.experimental.pallas import tpu_sc as plsc

B = 4096
E = 200
EP = 256
H = 50
HP = 52
L = 16
NC, NS = 2, 16
NW = NC * NS
BPW = B // NW
NB0 = 8
NB1 = 5
NBIAS = 782
GM = 3.5


def _splat1(ref, i):
    return plsc.load_gather(ref, [jnp.full((L,), i, jnp.int32)])


def _svdpp_sc(user2, movie2, uidx1, midx1, mr2, sq, iku, ikm, Bu2, Bi2,
              Pt, Qt, Yt):
    mesh = plsc.VectorSubcoreMesh(core_axis_name="c", subcore_axis_name="s")
    cp = pltpu.CompilerParams(
        needs_layout_passes=False, use_tc_tiling_on_sc=True
    )

    @functools.partial(
        pl.kernel,
        out_type=jax.ShapeDtypeStruct((B,), jnp.float32),
        mesh=mesh,
        compiler_params=cp,
        scratch_types=[
            pltpu.VMEM((2 * BPW,), jnp.int32),
            pltpu.VMEM((2 * BPW,), jnp.int32),
            pltpu.VMEM((BPW,), jnp.int32),
            pltpu.VMEM((BPW,), jnp.int32),
            pltpu.VMEM((BPW * 2 * HP,), jnp.int32),
            pltpu.VMEM((BPW,), jnp.float32),
            pltpu.VMEM((BPW,), jnp.float32),
            pltpu.VMEM((BPW,), jnp.float32),
            pltpu.VMEM((BPW,), jnp.int32),
            pltpu.VMEM((BPW, 128), jnp.float32),
            pltpu.VMEM((BPW,), jnp.float32),
            pltpu.VMEM((BPW,), jnp.float32),
            pltpu.VMEM((2 * BPW, 128), jnp.float32),
            pltpu.VMEM((2 * BPW, 128), jnp.float32),
            pltpu.VMEM((2 * HP, 128), jnp.float32),
            pltpu.VMEM((2 * HP, 128), jnp.float32),
            pltpu.VMEM((BPW,), jnp.float32),
            pltpu.SemaphoreType.DMA,
            pltpu.SemaphoreType.DMA,
            pltpu.SemaphoreType.DMA,
        ],
    )
    def kern(user2_h, movie2_h, uidx_h, midx_h, mr_h, sq_h, iku_h, ikm_h,
             bu_h, bi_h, p_h, q_h, y_h, out_h, u2idx, m2idx, uidx, midx,
             mr_v, sq_v, iku_v, ikm_v, rowi, biasbuf, bu_val, bi_val, p_v,
             q_v, ybuf0, ybuf1, outv, sem_pre, sem0, sem1):
        wid = lax.axis_index("s") * NC + lax.axis_index("c")
        base = wid * BPW

        pltpu.sync_copy(user2_h.at[pl.ds(2 * base, 2 * BPW)], u2idx)
        pltpu.sync_copy(movie2_h.at[pl.ds(2 * base, 2 * BPW)], m2idx)
        pltpu.sync_copy(uidx_h.at[pl.ds(base, BPW)], uidx)
        pltpu.sync_copy(midx_h.at[pl.ds(base, BPW)], midx)
        pltpu.sync_copy(mr_h.at[pl.ds(base * 2 * HP, BPW * 2 * HP)], mr_v)
        pltpu.sync_copy(sq_h.at[pl.ds(base, BPW)], sq_v)
        pltpu.sync_copy(iku_h.at[pl.ds(base, BPW)], iku_v)
        pltpu.sync_copy(ikm_h.at[pl.ds(base, BPW)], ikm_v)

        hp0 = pltpu.async_copy(
            p_h.at[u2idx.at[pl.ds(0, BPW)]], p_v.at[pl.ds(0, BPW)], sem_pre
        )
        hp1 = pltpu.async_copy(
            p_h.at[u2idx.at[pl.ds(BPW, BPW)]], p_v.at[pl.ds(BPW, BPW)],
            sem_pre
        )
        hq0 = pltpu.async_copy(
            q_h.at[m2idx.at[pl.ds(0, BPW)]], q_v.at[pl.ds(0, BPW)], sem_pre
        )
        hq1 = pltpu.async_copy(
            q_h.at[m2idx.at[pl.ds(BPW, BPW)]], q_v.at[pl.ds(BPW, BPW)],
            sem_pre
        )

        pltpu.async_copy(y_h.at[mr_v.at[pl.ds(0, 2 * HP)]], ybuf0, sem0)

        lane = lax.iota(jnp.int32, L)
        zero = jnp.zeros((L,), jnp.float32)

        def bias_values(idx_v, table_h, val_v):
            @pl.loop(0, BPW, step=L)
            def _(t):
                rowi[pl.ds(t, L)] = lax.shift_right_logical(
                    idx_v[pl.ds(t, L)], 7
                )

            pltpu.async_copy(table_h.at[rowi], biasbuf, sem_pre).wait()

            @pl.loop(0, BPW, step=L)
            def _(t):
                lanes = jnp.bitwise_and(idx_v[pl.ds(t, L)], 127)
                val_v[pl.ds(t, L)] = plsc.load_gather(
                    biasbuf, [lane + t, lanes]
                )

        bias_values(uidx, bu_h, bu_val)
        bias_values(midx, bi_h, bi_val)

        hp0.wait()
        hp1.wait()
        hq0.wait()
        hq1.wait()

        def compute(b, ybuf):
            def row_body(h, accs):
                even = tuple(
                    accs[j] + ybuf[2 * h, pl.ds(16 * j, L)]
                    for j in range(NB0)
                )
                odd = tuple(
                    accs[NB0 + j] + ybuf[2 * h + 1, pl.ds(16 * j, L)]
                    for j in range(NB1)
                )
                return even + odd

            accs = lax.fori_loop(0, H, row_body, (zero,) * (NB0 + NB1))

            iku_s = _splat1(iku_v, b)
            ikm_s = _splat1(ikm_v, b)
            sq_s = _splat1(sq_v, b)
            ysc = iku_s / sq_s
            tsum = zero
            for j in range(NB0):
                pj = p_v[2 * b, pl.ds(16 * j, L)]
                qj = q_v[2 * b, pl.ds(16 * j, L)]
                tsum = tsum + qj * (pj * iku_s + accs[j] * ysc)
            for j in range(NB1):
                pj = p_v[2 * b + 1, pl.ds(16 * j, L)]
                qj = q_v[2 * b + 1, pl.ds(16 * j, L)]
                tsum = tsum + qj * (pj * iku_s + accs[NB0 + j] * ysc)
            dot = jnp.sum(tsum)
            bu_s = _splat1(bu_val, b)
            bi_s = _splat1(bi_val, b)
            r = ikm_s * jnp.full((L,), dot, jnp.float32) \
                + bi_s * ikm_s + bu_s * iku_s + GM
            plsc.store_scatter(
                outv, [jnp.full((L,), b, jnp.int32)], r, mask=(lane == 0)
            )

        NH = 2 * HP

        @pl.loop(0, BPW, step=2)
        def _(g):
            pltpu.async_copy(
                y_h.at[mr_v.at[pl.ds((g + 1) * NH, NH)]], ybuf1, sem1
            )
            pltpu.make_async_copy(
                y_h.at[mr_v.at[pl.ds(g * NH, NH)]], ybuf0, sem0
            ).wait()
            compute(g, ybuf0)

            @pl.when(g + 2 < BPW)
            def _():
                pltpu.async_copy(
                    y_h.at[mr_v.at[pl.ds((g + 2) * NH, NH)]], ybuf0, sem0
                )

            pltpu.make_async_copy(
                y_h.at[mr_v.at[pl.ds((g + 1) * NH, NH)]], ybuf1, sem1
            ).wait()
            compute(g + 1, ybuf1)

        pltpu.sync_copy(outv, out_h.at[pl.ds(base, BPW)])

    return kern(user2, movie2, uidx1, midx1, mr2, sq, iku, ikm, Bu2, Bi2,
                Pt, Qt, Yt)


def _halfrows(table, s):
    n = table.shape[0]
    padded = jnp.pad(table, ((0, 0), (0, EP - E))) * s
    return padded.reshape(2 * n, 128)


@jax.jit
def kernel(user, movie, movies_rated_by_this_user, users_who_rated_this_movie,
           sqrt_of_number_of_movies_rated_by_this_user,
           sqrt_of_number_of_users_who_rated_this_movie,
           is_known_user, is_known_movie, Bu, Bi, P, Q, Y):
    del users_who_rated_this_movie, sqrt_of_number_of_users_who_rated_this_movie
    sq = sqrt_of_number_of_movies_rated_by_this_user.reshape(B)
    iku = is_known_user.reshape(B)
    ikm = is_known_movie.reshape(B)

    s = lax.optimization_barrier(jnp.float32(1.0))
    Pt = _halfrows(P, s)
    Qt = _halfrows(Q, s)
    Yt = _halfrows(Y, s)

    two = jnp.arange(2, dtype=jnp.int32)
    user_i = user.astype(jnp.int32)
    movie_i = movie.astype(jnp.int32)
    user2 = (2 * user_i[:, None] + two).reshape(2 * B)
    movie2 = (2 * movie_i[:, None] + two).reshape(2 * B)

    mr_i = movies_rated_by_this_user.astype(jnp.int32)
    mr_p = jnp.pad(mr_i, ((0, 0), (0, HP - H)))
    mr2 = (2 * mr_p[:, :, None] + two).reshape(B * 2 * HP)

    Bu2 = (jnp.pad(Bu.reshape(-1), (0, NBIAS * 128 - Bu.shape[0])) * s
           ).reshape(NBIAS, 128)
    Bi2 = (jnp.pad(Bi.reshape(-1), (0, NBIAS * 128 - Bi.shape[0])) * s
           ).reshape(NBIAS, 128)

    return _svdpp_sc(user2, movie2, user_i, movie_i, mr2, sq, iku, ikm,
                     Bu2, Bi2, Pt, Qt, Yt)


# speedup vs baseline: 2.0773x; 1.6066x over previous
import functools

import jax
import jax.numpy as jnp
from jax import lax
from jax.experimental import pallas as pl
from jax.experimental.pallas import tpu as pltpu
from jax.experimental.pallas import tpu_sc as plsc

B = 4096
E = 200
H = 50
HP = 56
L = 16
NC, NS = 2, 16
NW = NC * NS
BPW = B // NW
NB0 = 8
NB1 = 5
NP = 100096
NBIAS = NP // 128
TCOL = 2176
TGRID = NP // TCOL
GM = 3.5


def _splat1(ref, i):
    return plsc.load_gather(ref, [jnp.full((L,), i, jnp.int32)])


def _halfrow_table(x):
    xt = x.T

    def body(x_ref, o_ref):
        o_ref[...] = x_ref[...].T

    return pl.pallas_call(
        body,
        grid=(2, TGRID),
        in_specs=[
            pl.BlockSpec((128, TCOL), lambda s, i: (s, i)),
        ],
        out_specs=pl.BlockSpec((TCOL, 128), lambda s, i: (s * TGRID + i, 0)),
        out_shape=jax.ShapeDtypeStruct((2 * NP, 128), jnp.float32),
    )(xt)


def _svdpp_sc(userA, movieA, userB, movieB, mrA, mrB, sq, iku, ikm,
              Bu2, Bi2, Pt, Qt, Yt):
    mesh = plsc.VectorSubcoreMesh(core_axis_name="c", subcore_axis_name="s")
    cp = pltpu.CompilerParams(
        needs_layout_passes=False, use_tc_tiling_on_sc=True
    )

    @functools.partial(
        pl.kernel,
        out_type=jax.ShapeDtypeStruct((B,), jnp.float32),
        mesh=mesh,
        compiler_params=cp,
        scratch_types=[
            pltpu.VMEM((BPW,), jnp.int32),
            pltpu.VMEM((BPW,), jnp.int32),
            pltpu.VMEM((BPW,), jnp.int32),
            pltpu.VMEM((BPW,), jnp.int32),
            pltpu.VMEM((BPW * HP,), jnp.int32),
            pltpu.VMEM((BPW * HP,), jnp.int32),
            pltpu.VMEM((BPW,), jnp.float32),
            pltpu.VMEM((BPW,), jnp.float32),
            pltpu.VMEM((BPW,), jnp.float32),
            pltpu.VMEM((BPW,), jnp.int32),
            pltpu.VMEM((BPW, 128), jnp.float32),
            pltpu.VMEM((BPW,), jnp.float32),
            pltpu.VMEM((BPW,), jnp.float32),
            pltpu.VMEM((BPW, 128), jnp.float32),
            pltpu.VMEM((BPW, 128), jnp.float32),
            pltpu.VMEM((BPW, 128), jnp.float32),
            pltpu.VMEM((BPW, 128), jnp.float32),
            pltpu.VMEM((HP, 128), jnp.float32),
            pltpu.VMEM((HP, 128), jnp.float32),
            pltpu.VMEM((HP, 128), jnp.float32),
            pltpu.VMEM((HP, 128), jnp.float32),
            pltpu.VMEM((BPW,), jnp.float32),
            pltpu.SemaphoreType.DMA,
            pltpu.SemaphoreType.DMA,
            pltpu.SemaphoreType.DMA,
        ],
    )
    def kern(uA_h, mA_h, uB_h, mB_h, mrA_h, mrB_h, sq_h, iku_h, ikm_h,
             bu_h, bi_h, p_h, q_h, y_h, out_h, uAi, mAi, uBi, mBi, mrA_v,
             mrB_v, sq_v, iku_v, ikm_v, rowi, biasbuf, bu_val, bi_val,
             pA, pB, qA, qB, yA0, yB0, yA1, yB1, outv, sem_pre, sem0,
             sem1):
        wid = lax.axis_index("s") * NC + lax.axis_index("c")
        base = wid * BPW

        pltpu.sync_copy(uA_h.at[pl.ds(base, BPW)], uAi)
        pltpu.sync_copy(mA_h.at[pl.ds(base, BPW)], mAi)
        pltpu.sync_copy(uB_h.at[pl.ds(base, BPW)], uBi)
        pltpu.sync_copy(mB_h.at[pl.ds(base, BPW)], mBi)
        pltpu.sync_copy(mrA_h.at[pl.ds(base * HP, BPW * HP)], mrA_v)
        pltpu.sync_copy(mrB_h.at[pl.ds(base * HP, BPW * HP)], mrB_v)
        pltpu.sync_copy(sq_h.at[pl.ds(base, BPW)], sq_v)
        pltpu.sync_copy(iku_h.at[pl.ds(base, BPW)], iku_v)
        pltpu.sync_copy(ikm_h.at[pl.ds(base, BPW)], ikm_v)

        hp0 = pltpu.async_copy(p_h.at[uAi], pA, sem_pre)
        hp1 = pltpu.async_copy(p_h.at[uBi], pB, sem_pre)
        hq0 = pltpu.async_copy(q_h.at[mAi], qA, sem_pre)
        hq1 = pltpu.async_copy(q_h.at[mBi], qB, sem_pre)

        pltpu.async_copy(y_h.at[mrA_v.at[pl.ds(0, HP)]], yA0, sem0)
        pltpu.async_copy(y_h.at[mrB_v.at[pl.ds(0, HP)]], yB0, sem0)

        lane = lax.iota(jnp.int32, L)
        tail_mask = lane < (E - 128 - 16 * (NB1 - 1))
        zero = jnp.zeros((L,), jnp.float32)

        def bias_values(idx_v, table_h, val_v):
            @pl.loop(0, BPW, step=L)
            def _(t):
                rowi[pl.ds(t, L)] = lax.shift_right_logical(
                    idx_v[pl.ds(t, L)], 7
                )

            pltpu.async_copy(table_h.at[rowi], biasbuf, sem_pre).wait()

            @pl.loop(0, BPW, step=L)
            def _(t):
                lanes = jnp.bitwise_and(idx_v[pl.ds(t, L)], 127)
                val_v[pl.ds(t, L)] = plsc.load_gather(
                    biasbuf, [lane + t, lanes]
                )

        bias_values(uAi, bu_h, bu_val)
        bias_values(mAi, bi_h, bi_val)

        hp0.wait()
        hp1.wait()
        hq0.wait()
        hq1.wait()

        def compute(b, ybA, ybB):
            def row_body(h, accs):
                even = tuple(
                    accs[j] + ybA[h, pl.ds(16 * j, L)] for j in range(NB0)
                )
                odd = tuple(
                    accs[NB0 + j] + ybB[h, pl.ds(16 * j, L)]
                    for j in range(NB1)
                )
                return even + odd

            accs = lax.fori_loop(0, H, row_body, (zero,) * (NB0 + NB1))

            iku_s = _splat1(iku_v, b)
            ikm_s = _splat1(ikm_v, b)
            sq_s = _splat1(sq_v, b)
            ysc = iku_s / sq_s
            tsum = zero
            for j in range(NB0):
                pj = pA[b, pl.ds(16 * j, L)]
                qj = qA[b, pl.ds(16 * j, L)]
                tsum = tsum + qj * (pj * iku_s + accs[j] * ysc)
            for j in range(NB1 - 1):
                pj = pB[b, pl.ds(16 * j, L)]
                qj = qB[b, pl.ds(16 * j, L)]
                tsum = tsum + qj * (pj * iku_s + accs[NB0 + j] * ysc)
            pj = pB[b, pl.ds(16 * (NB1 - 1), L)]
            qj = qB[b, pl.ds(16 * (NB1 - 1), L)]
            tt = qj * (pj * iku_s + accs[NB0 + NB1 - 1] * ysc)
            tsum = tsum + jnp.where(tail_mask, tt, zero)
            dot = jnp.sum(tsum)
            bu_s = _splat1(bu_val, b)
            bi_s = _splat1(bi_val, b)
            r = ikm_s * jnp.full((L,), dot, jnp.float32) \
                + bi_s * ikm_s + bu_s * iku_s + GM
            plsc.store_scatter(
                outv, [jnp.full((L,), b, jnp.int32)], r, mask=(lane == 0)
            )

        @pl.loop(0, BPW, step=2)
        def _(g):
            pltpu.async_copy(
                y_h.at[mrA_v.at[pl.ds((g + 1) * HP, HP)]], yA1, sem1
            )
            pltpu.async_copy(
                y_h.at[mrB_v.at[pl.ds((g + 1) * HP, HP)]], yB1, sem1
            )
            pltpu.make_async_copy(
                y_h.at[mrA_v.at[pl.ds(g * HP, HP)]], yA0, sem0
            ).wait()
            pltpu.make_async_copy(
                y_h.at[mrB_v.at[pl.ds(g * HP, HP)]], yB0, sem0
            ).wait()
            compute(g, yA0, yB0)

            @pl.when(g + 2 < BPW)
            def _():
                pltpu.async_copy(
                    y_h.at[mrA_v.at[pl.ds((g + 2) * HP, HP)]], yA0, sem0
                )
                pltpu.async_copy(
                    y_h.at[mrB_v.at[pl.ds((g + 2) * HP, HP)]], yB0, sem0
                )

            pltpu.make_async_copy(
                y_h.at[mrA_v.at[pl.ds((g + 1) * HP, HP)]], yA1, sem1
            ).wait()
            pltpu.make_async_copy(
                y_h.at[mrB_v.at[pl.ds((g + 1) * HP, HP)]], yB1, sem1
            ).wait()
            compute(g + 1, yA1, yB1)

        pltpu.sync_copy(outv, out_h.at[pl.ds(base, BPW)])

    return kern(userA, movieA, userB, movieB, mrA, mrB, sq, iku, ikm,
                Bu2, Bi2, Pt, Qt, Yt)


@jax.jit
def kernel(user, movie, movies_rated_by_this_user, users_who_rated_this_movie,
           sqrt_of_number_of_movies_rated_by_this_user,
           sqrt_of_number_of_users_who_rated_this_movie,
           is_known_user, is_known_movie, Bu, Bi, P, Q, Y):
    del users_who_rated_this_movie, sqrt_of_number_of_users_who_rated_this_movie
    sq = sqrt_of_number_of_movies_rated_by_this_user.reshape(B)
    iku = is_known_user.reshape(B)
    ikm = is_known_movie.reshape(B)

    Pt = _halfrow_table(P)
    Qt = _halfrow_table(Q)
    Yt = _halfrow_table(Y)

    user_i = user.astype(jnp.int32)
    movie_i = movie.astype(jnp.int32)
    userB_i = user_i + NP
    movieB_i = movie_i + NP

    mr_i = movies_rated_by_this_user.astype(jnp.int32)
    mr_p = jnp.pad(mr_i, ((0, 0), (0, HP - H)))
    mrA = mr_p.reshape(B * HP)
    mrB = (mr_p + NP).reshape(B * HP)

    Bu2 = jnp.pad(Bu.reshape(-1), (0, NP - Bu.shape[0])).reshape(NBIAS, 128)
    Bi2 = jnp.pad(Bi.reshape(-1), (0, NP - Bi.shape[0])).reshape(NBIAS, 128)

    return _svdpp_sc(user_i, movie_i, userB_i, movieB_i, mrA, mrB,
                     sq, iku, ikm, Bu2, Bi2, Pt, Qt, Yt)
